# baseline (speedup 1.0000x reference)
import functools

import jax
import jax.numpy as jnp
from jax import lax
from jax.experimental import pallas as pl
from jax.experimental.pallas import tpu as pltpu
from jax.experimental.pallas import tpu_sc as plsc

DIM = 1024
HEADS = 16
EXPERTS = 8
BETA = 0.5
TOPK = 3
D_H = DIM // HEADS
R_H = 8
CAT = EXPERTS * R_H

TB = 1024
SPLIT1 = 5


def _sc_gate(ref, base):
    x = [ref[base + e] for e in range(EXPERTS)]
    m = x[0]
    for e in range(1, EXPERTS):
        m = jnp.maximum(m, x[e])
    ee = [jnp.exp(x[e] - m) for e in range(EXPERTS)]
    s = ee[0]
    for e in range(1, EXPERTS):
        s = s + ee[e]
    p = [ee[e] / s for e in range(EXPERTS)]
    one = jnp.ones((HEADS,), jnp.float32)
    zero = jnp.zeros((HEADS,), jnp.float32)
    sel = []
    for e in range(EXPERTS):
        rank = zero
        for e2 in range(EXPERTS):
            if e2 == e:
                continue
            beats = (p[e2] >= p[e]) if e2 < e else (p[e2] > p[e])
            rank = rank + jnp.where(beats, one, zero)
        sel.append(jnp.where(rank < float(TOPK), p[e], zero))
    d = sel[0]
    for e in range(1, EXPERTS):
        d = d + sel[e]
    d = d + 1e-8
    return [sel[e] / d for e in range(EXPERTS)]


def _sc_router(zcat_t):
    mesh = plsc.VectorSubcoreMesh(core_axis_name="c", subcore_axis_name="s",
                                  num_cores=1)

    @functools.partial(
        pl.kernel,
        mesh=mesh,
        out_type=jax.ShapeDtypeStruct((CAT, HEADS), jnp.float32),
        scratch_types=[
            pltpu.VMEM((2 * EXPERTS, HEADS), jnp.float32),
            pltpu.VMEM((CAT, HEADS), jnp.float32),
        ],
    )
    def k(zcat_hbm, out_hbm, zcat_v, out_v):
        first = (lax.axis_index("c") == 0) & (lax.axis_index("s") == 0)

        @pl.when(first)
        def _():
            pltpu.sync_copy(zcat_hbm, zcat_v)
            ws = _sc_gate(zcat_v, 0)
            wa = _sc_gate(zcat_v, EXPERTS)
            for e in range(EXPERTS):
                wtot_e = ws[e] + wa[e]
                for r in range(R_H):
                    out_v[e * R_H + r] = wtot_e
            pltpu.sync_copy(out_v, out_hbm)

    return k(zcat_t)


def _tc_gate(zr):
    m = jnp.max(zr, axis=-1, keepdims=True)
    e = jnp.exp(zr - m)
    p = e / jnp.sum(e, axis=-1, keepdims=True)
    pa = p[:, :, None]
    pb = p[:, None, :]
    ia = jax.lax.broadcasted_iota(jnp.int32, (HEADS, EXPERTS, EXPERTS), 1)
    ib = jax.lax.broadcasted_iota(jnp.int32, (HEADS, EXPERTS, EXPERTS), 2)
    beats = (pb > pa) | ((pb == pa) & (ib < ia))
    rank = jnp.sum(beats.astype(jnp.float32), axis=2)
    sel = jnp.where(rank < float(TOPK), p, 0.0)
    return sel / (jnp.sum(sel, axis=-1, keepdims=True) + 1e-8)


def _delta_apply(wrept, acat, bcat, z, out_ref):
    zb = z.astype(jnp.bfloat16)
    for p in range(HEADS // 2):
        ma = (BETA * jnp.dot(acat, bcat * wrept[:, 2 * p:2 * p + 1],
                             preferred_element_type=jnp.float32)
              ).astype(jnp.bfloat16)
        mb = (BETA * jnp.dot(acat, bcat * wrept[:, 2 * p + 1:2 * p + 2],
                             preferred_element_type=jnp.float32)
              ).astype(jnp.bfloat16)
        zero = jnp.zeros((D_H, D_H), dtype=jnp.bfloat16)
        wpair = jnp.concatenate(
            [jnp.concatenate([ma, zero], axis=1),
             jnp.concatenate([zero, mb], axis=1)], axis=0)
        sl = slice(p * 2 * D_H, (p + 1) * 2 * D_H)
        out_ref[:, sl] = z[:, sl] + jnp.dot(zb[:, sl], wpair,
                                            preferred_element_type=jnp.float32)


def _apply_selfgate_kernel(zs_ref, za_ref, acat_ref, bcat_ref, z_ref, out_ref):
    wtot = _tc_gate(zs_ref[...]) + _tc_gate(za_ref[...])
    re_ = jax.lax.broadcasted_iota(jnp.int32, (CAT, EXPERTS), 0)
    rc = jax.lax.broadcasted_iota(jnp.int32, (CAT, EXPERTS), 1)
    sel_mat_t = (re_ // R_H == rc).astype(jnp.float32)
    wrept = jnp.dot(sel_mat_t, wtot.T,
                    preferred_element_type=jnp.float32)
    _delta_apply(wrept, acat_ref[...], bcat_ref[...], z_ref[...], out_ref)


def _apply_scgate_kernel(wrept_ref, acat_ref, bcat_ref, z_ref, prev_ref,
                         out_ref):
    del prev_ref
    _delta_apply(wrept_ref[...], acat_ref[...], bcat_ref[...], z_ref[...],
                 out_ref)


def kernel(z, A, B_mat, Z_share, Z_auth):
    batch = z.shape[0]
    nblk = batch // TB
    n1 = SPLIT1
    n2 = nblk - n1
    acat = jnp.transpose(A, (1, 0, 2)).reshape(D_H, CAT)
    bcat = B_mat.reshape(CAT, D_H)

    wrept = _sc_router(jnp.concatenate([Z_share.T, Z_auth.T], axis=0))

    out1 = pl.pallas_call(
        _apply_selfgate_kernel,
        grid=(n1,),
        in_specs=[
            pl.BlockSpec((HEADS, EXPERTS), lambda i: (0, 0)),
            pl.BlockSpec((HEADS, EXPERTS), lambda i: (0, 0)),
            pl.BlockSpec((D_H, CAT), lambda i: (0, 0)),
            pl.BlockSpec((CAT, D_H), lambda i: (0, 0)),
            pl.BlockSpec((TB, DIM), lambda i: (i, 0)),
        ],
        out_specs=pl.BlockSpec((TB, DIM), lambda i: (i, 0)),
        out_shape=jax.ShapeDtypeStruct((batch, DIM), jnp.float32),
        compiler_params=pltpu.CompilerParams(
            dimension_semantics=("parallel",),
        ),
    )(Z_share, Z_auth, acat, bcat, z)

    out = pl.pallas_call(
        _apply_scgate_kernel,
        grid=(n2,),
        in_specs=[
            pl.BlockSpec((CAT, HEADS), lambda i: (0, 0)),
            pl.BlockSpec((D_H, CAT), lambda i: (0, 0)),
            pl.BlockSpec((CAT, D_H), lambda i: (0, 0)),
            pl.BlockSpec((TB, DIM), lambda i: (i + SPLIT1, 0)),
            pl.BlockSpec(memory_space=pl.ANY),
        ],
        out_specs=pl.BlockSpec((TB, DIM), lambda i: (i + SPLIT1, 0)),
        out_shape=jax.ShapeDtypeStruct((batch, DIM), jnp.float32),
        input_output_aliases={4: 0},
        compiler_params=pltpu.CompilerParams(
            dimension_semantics=("parallel",),
        ),
    )(wrept, acat, bcat, z, out1)
    return out


# speedup vs baseline: 1.1617x; 1.1617x over previous
import functools

import jax
import jax.numpy as jnp
from jax import lax
from jax.experimental import pallas as pl
from jax.experimental.pallas import tpu as pltpu
from jax.experimental.pallas import tpu_sc as plsc

DIM = 1024
HEADS = 16
EXPERTS = 8
BETA = 0.5
TOPK = 3
D_H = DIM // HEADS
R_H = 8
CAT = EXPERTS * R_H

TB = 2048


def _sc_gate(ref, base):
    x = [ref[base + e] for e in range(EXPERTS)]
    m = x[0]
    for e in range(1, EXPERTS):
        m = jnp.maximum(m, x[e])
    ee = [jnp.exp(x[e] - m) for e in range(EXPERTS)]
    s = ee[0]
    for e in range(1, EXPERTS):
        s = s + ee[e]
    p = [ee[e] / s for e in range(EXPERTS)]
    one = jnp.ones((HEADS,), jnp.float32)
    zero = jnp.zeros((HEADS,), jnp.float32)
    sel = []
    for e in range(EXPERTS):
        rank = zero
        for e2 in range(EXPERTS):
            if e2 == e:
                continue
            beats = (p[e2] >= p[e]) if e2 < e else (p[e2] > p[e])
            rank = rank + jnp.where(beats, one, zero)
        sel.append(jnp.where(rank < float(TOPK), p[e], zero))
    d = sel[0]
    for e in range(1, EXPERTS):
        d = d + sel[e]
    d = d + 1e-8
    return [sel[e] / d for e in range(EXPERTS)]


def _sc_router(zcat_t):
    mesh = plsc.VectorSubcoreMesh(core_axis_name="c", subcore_axis_name="s",
                                  num_cores=1)

    @functools.partial(
        pl.kernel,
        mesh=mesh,
        out_type=jax.ShapeDtypeStruct((EXPERTS, HEADS), jnp.float32),
        scratch_types=[
            pltpu.VMEM((2 * EXPERTS, HEADS), jnp.float32),
            pltpu.VMEM((EXPERTS, HEADS), jnp.float32),
        ],
    )
    def k(zcat_hbm, out_hbm, zcat_v, out_v):
        first = (lax.axis_index("c") == 0) & (lax.axis_index("s") == 0)

        @pl.when(first)
        def _():
            pltpu.sync_copy(zcat_hbm, zcat_v)
            ws = _sc_gate(zcat_v, 0)
            wa = _sc_gate(zcat_v, EXPERTS)
            for e in range(EXPERTS):
                out_v[e] = ws[e] + wa[e]
            pltpu.sync_copy(out_v, out_hbm)

    return k(zcat_t)


def _apply_kernel(wtott_ref, acat_ref, bcat_ref, z_ref, out_ref):
    acat = acat_ref[...]
    bcat = bcat_ref[...]
    re_ = jax.lax.broadcasted_iota(jnp.int32, (CAT, EXPERTS), 0)
    rc = jax.lax.broadcasted_iota(jnp.int32, (CAT, EXPERTS), 1)
    sel_mat_t = (re_ // R_H == rc).astype(jnp.float32)
    wrept = jnp.dot(sel_mat_t, wtott_ref[...],
                    preferred_element_type=jnp.float32)
    z = z_ref[...]
    zb = z.astype(jnp.bfloat16)
    for p in range(HEADS // 2):
        ma = (BETA * jnp.dot(acat, bcat * wrept[:, 2 * p:2 * p + 1],
                             preferred_element_type=jnp.float32)
              ).astype(jnp.bfloat16)
        mb = (BETA * jnp.dot(acat, bcat * wrept[:, 2 * p + 1:2 * p + 2],
                             preferred_element_type=jnp.float32)
              ).astype(jnp.bfloat16)
        zero = jnp.zeros((D_H, D_H), dtype=jnp.bfloat16)
        wpair = jnp.concatenate(
            [jnp.concatenate([ma, zero], axis=1),
             jnp.concatenate([zero, mb], axis=1)], axis=0)
        sl = slice(p * 2 * D_H, (p + 1) * 2 * D_H)
        out_ref[:, sl] = z[:, sl] + jnp.dot(zb[:, sl], wpair,
                                            preferred_element_type=jnp.float32)


def kernel(z, A, B_mat, Z_share, Z_auth):
    batch = z.shape[0]
    acat = jnp.transpose(A, (1, 0, 2)).reshape(D_H, CAT)
    bcat = B_mat.reshape(CAT, D_H)

    wtott = _sc_router(jnp.concatenate([Z_share.T, Z_auth.T], axis=0))

    out = pl.pallas_call(
        _apply_kernel,
        grid=(batch // TB,),
        in_specs=[
            pl.BlockSpec((EXPERTS, HEADS), lambda i: (0, 0)),
            pl.BlockSpec((D_H, CAT), lambda i: (0, 0)),
            pl.BlockSpec((CAT, D_H), lambda i: (0, 0)),
            pl.BlockSpec((TB, DIM), lambda i: (i, 0)),
        ],
        out_specs=pl.BlockSpec((TB, DIM), lambda i: (i, 0)),
        out_shape=jax.ShapeDtypeStruct((batch, DIM), jnp.float32),
        compiler_params=pltpu.CompilerParams(
            dimension_semantics=("parallel",),
        ),
    )(wtott, acat, bcat, z)
    return out
